# Initial kernel scaffold; baseline (speedup 1.0000x reference)
import jax
import jax.numpy as jnp
from jax.experimental import pallas as pl


def kernel(q, k, v, mask, Wq, bq, Wk, bk, Wv, bv):
    raise NotImplementedError("write your pallas kernel here")


# speedup vs baseline: 4.5155x; 4.5155x over previous
import math

import jax
import jax.numpy as jnp
import numpy as np
from jax.experimental import pallas as pl
from jax.experimental.pallas import tpu as pltpu

_D = 1024
_H = 16
_E = _D // _H
_L = 2048
_C_FACTOR = 5
_SAMPLE_K = _C_FACTOR * int(math.ceil(math.log(_L)))
_N_TOP = _C_FACTOR * int(math.ceil(math.log(_L)))
_NEG = float(-(2 ** 32) + 1)
_TM = 512


def _sample_count_matrix_t():
    idx = np.asarray(
        jax.random.randint(jax.random.key(42), (_L, _SAMPLE_K), 0, _L)
    )
    c = np.zeros((_L, _L), np.float32)
    np.add.at(c, (np.arange(_L)[:, None], idx), 1.0)
    return np.ascontiguousarray(c.T)


_CT_NP = _sample_count_matrix_t()


def _proj3_body(q_ref, k_ref, v_ref, wq_ref, bq_ref, wk_ref, bk_ref,
                wv_ref, bv_ref, qo_ref, ko_ref, vo_ref):
    qo_ref[...] = (
        jnp.dot(q_ref[...], wq_ref[...], preferred_element_type=jnp.float32)
        + bq_ref[...]
    )
    ko_ref[...] = (
        jnp.dot(k_ref[...], wk_ref[...], preferred_element_type=jnp.float32)
        + bk_ref[...]
    )
    vo_ref[...] = (
        jnp.dot(v_ref[...], wv_ref[...], preferred_element_type=jnp.float32)
        + bv_ref[...]
    )


def _head_body(qh_ref, kh_ref, vh_ref, ct_ref, o_ref, qr_scr):
    kh = kh_ref[0]

    m_rows = []
    for t in range(_L // 256):
        qt = qh_ref[0, t * 256:(t + 1) * 256, :]
        st = jax.lax.dot_general(
            kh, qt, (((1,), (1,)), ((), ())),
            preferred_element_type=jnp.float32,
        )
        ct = ct_ref[:, t * 256:(t + 1) * 256].astype(jnp.float32)
        ssum = jnp.sum(st * ct, axis=0, keepdims=True)
        smax = jnp.max(
            jnp.where(ct > 0.0, st, jnp.float32(-1e30)), axis=0, keepdims=True
        )
        m_rows.append(smax - ssum * (1.0 / _L))
    m = jnp.concatenate(m_rows, axis=0)

    qidx = (
        jax.lax.broadcasted_iota(jnp.int32, (8, 256), 0) * 256
        + jax.lax.broadcasted_iota(jnp.int32, (8, 256), 1)
    )
    qr_scr[...] = jnp.zeros((64, _E), jnp.float32)
    th = jnp.full((64, 1), jnp.int32(_L))
    row64 = jax.lax.broadcasted_iota(jnp.int32, (64, 1), 0)
    idxs = []
    for j in range(_N_TOP):
        mx = jnp.max(m)
        cand = jnp.where(m == mx, qidx, jnp.int32(2 ** 30))
        ii = jnp.min(cand)
        idxs.append(ii)
        m = jnp.where(qidx == ii, jnp.float32(-1e30), m)
        th = jnp.where(row64 == j, ii, th)
        qr_scr[j:j + 1, :] = qh_ref[0, pl.ds(ii, 1), :]

    scores = jax.lax.dot_general(
        qr_scr[...], kh, (((1,), (1,)), ((), ())),
        preferred_element_type=jnp.float32,
    ) * (1.0 / math.sqrt(_E))
    kcol = jax.lax.broadcasted_iota(jnp.int32, (64, _L), 1)
    scores = jnp.where(kcol <= th, scores, jnp.float32(_NEG))
    smax = jnp.max(scores, axis=1, keepdims=True)
    p = jnp.exp(scores - smax)
    p = p / jnp.sum(p, axis=1, keepdims=True)
    out_sel = jnp.dot(p, vh_ref[0], preferred_element_type=jnp.float32)

    r256 = jax.lax.broadcasted_iota(jnp.int32, (256, 256), 0)
    c256 = jax.lax.broadcasted_iota(jnp.int32, (256, 256), 1)
    tril = jnp.where(r256 >= c256, jnp.float32(1.0), jnp.float32(0.0))
    carry = jnp.zeros((1, _E), jnp.float32)
    for t in range(_L // 256):
        vt = vh_ref[0, t * 256:(t + 1) * 256, :]
        o_ref[0, t * 256:(t + 1) * 256, :] = (
            jnp.dot(tril, vt, preferred_element_type=jnp.float32) + carry
        )
        carry = carry + jnp.sum(vt, axis=0, keepdims=True)

    for j in range(_N_TOP):
        o_ref[0, pl.ds(idxs[j], 1), :] = out_sel[j:j + 1, :]


def kernel(q, k, v, mask, Wq, bq, Wk, bk, Wv, bv):
    B, L, D = q.shape
    rows = B * L
    q2 = q.reshape(rows, D)
    k2 = k.reshape(rows, D)
    v2 = v.reshape(rows, D)

    mat_spec = pl.BlockSpec((D, D), lambda g: (0, 0))
    bias_spec = pl.BlockSpec((1, D), lambda g: (0, 0))
    row_spec = pl.BlockSpec((_TM, D), lambda g: (g, 0))
    qp, kp, vp = pl.pallas_call(
        _proj3_body,
        grid=(rows // _TM,),
        in_specs=[row_spec, row_spec, row_spec,
                  mat_spec, bias_spec, mat_spec, bias_spec,
                  mat_spec, bias_spec],
        out_specs=[row_spec, row_spec, row_spec],
        out_shape=[jax.ShapeDtypeStruct((rows, D), jnp.float32)] * 3,
    )(q2, k2, v2, Wq, bq.reshape(1, D), Wk, bk.reshape(1, D),
      Wv, bv.reshape(1, D))

    bh = B * _H
    qh = qp.reshape(bh, _L, _E)
    kh = kp.reshape(bh, _L, _E)
    vh = vp.reshape(bh, _L, _E)
    ct = jnp.asarray(_CT_NP, dtype=jnp.bfloat16)

    head_spec = pl.BlockSpec((1, _L, _E), lambda g: (g, 0, 0))
    ctx = pl.pallas_call(
        _head_body,
        grid=(bh,),
        in_specs=[head_spec, head_spec, head_spec,
                  pl.BlockSpec((_L, _L), lambda g: (0, 0))],
        out_specs=head_spec,
        out_shape=jax.ShapeDtypeStruct((bh, _L, _E), jnp.float32),
        scratch_shapes=[pltpu.VMEM((64, _E), jnp.float32)],
    )(qh, kh, vh, ct)

    return ctx.reshape(B, L, D)
